# baseline (device time: 176903 ns/iter reference)
import jax
import jax.numpy as jnp
from jax import lax
from jax.experimental import pallas as pl
from jax.experimental.pallas import tpu as pltpu

N_DEV = 8
N_FLOW = 8


def kernel(x, w_mat):
    m, _k_shard = x.shape
    n = w_mat.shape[1]
    m_blk = m // N_DEV
    nc = n // N_FLOW

    def body(x_ref, w_ref, out_ref, send_buf, recv_buf, ssem, rsem):
        me = lax.axis_index("i")
        left = lax.rem(me - 1 + N_DEV, N_DEV)
        right = lax.rem(me + 1, N_DEV)

        barrier_sem = pltpu.get_barrier_semaphore()
        for nbr in (left, right):
            pl.semaphore_signal(
                barrier_sem, inc=1, device_id=(nbr,),
                device_id_type=pl.DeviceIdType.MESH,
            )
        pl.semaphore_wait(barrier_sem, 2)

        def block_idx(t, cw):
            if cw:
                return lax.rem(me - 1 - t + 2 * N_DEV, N_DEV)
            return lax.rem(me + 1 + t, N_DEV)

        def flow_dot(t, f):
            b = block_idx(t, cw=f < N_FLOW // 2)
            return jnp.dot(
                x_ref[pl.ds(b * m_blk, m_blk), :],
                w_ref[:, f * nc:(f + 1) * nc],
                preferred_element_type=jnp.float32,
            )

        def make_rdma(t, f, slot):
            return pltpu.make_async_remote_copy(
                src_ref=send_buf.at[f, slot],
                dst_ref=recv_buf.at[f, t],
                send_sem=ssem.at[f, t],
                recv_sem=rsem.at[f, t],
                device_id=(right if f < N_FLOW // 2 else left,),
                device_id_type=pl.DeviceIdType.MESH,
            )

        flow_order = (0, 4, 1, 5, 2, 6, 3, 7)

        rdmas = {}
        for f in flow_order:
            p = flow_dot(0, f)
            send_buf[f, 0, :, :] = p
            rdmas[(0, f)] = make_rdma(0, f, 0)
            rdmas[(0, f)].start()

        for t in range(1, N_DEV):
            for f in flow_order:
                p = flow_dot(t, f)
                rdmas[(t - 1, f)].wait_recv()
                acc = p + recv_buf[f, t - 1]
                if t < N_DEV - 1:
                    slot = t % 2
                    if t >= 2:
                        rdmas[(t - 2, f)].wait_send()
                    send_buf[f, slot, :, :] = acc
                    rdmas[(t, f)] = make_rdma(t, f, slot)
                    rdmas[(t, f)].start()
                else:
                    out_ref[:, f * nc:(f + 1) * nc] = jnp.maximum(acc, 0.0)

        for t in (N_DEV - 3, N_DEV - 2):
            for f in flow_order:
                rdmas[(t, f)].wait_send()

    return pl.pallas_call(
        body,
        out_shape=jax.ShapeDtypeStruct((m_blk, n), jnp.float32),
        in_specs=[
            pl.BlockSpec(memory_space=pltpu.VMEM),
            pl.BlockSpec(memory_space=pltpu.VMEM),
        ],
        out_specs=pl.BlockSpec(memory_space=pltpu.VMEM),
        scratch_shapes=[
            pltpu.VMEM((N_FLOW, 2, m_blk, nc), jnp.float32),
            pltpu.VMEM((N_FLOW, N_DEV - 1, m_blk, nc), jnp.float32),
            pltpu.SemaphoreType.DMA((N_FLOW, N_DEV - 1)),
            pltpu.SemaphoreType.DMA((N_FLOW, N_DEV - 1)),
        ],
        compiler_params=pltpu.CompilerParams(
            collective_id=0,
            vmem_limit_bytes=100 * 1024 * 1024,
        ),
    )(x, w_mat)


# device time: 140925 ns/iter; 1.2553x vs baseline; 1.2553x over previous
import jax
import jax.numpy as jnp
from jax import lax
from jax.experimental import pallas as pl
from jax.experimental.pallas import tpu as pltpu

N_DEV = 8
M_BLK = 512
COLS = (640, 640, 768)
OFFS = (0, 640, 1280)
PERMS = ((0, 1, 2), (1, 2, 0), (2, 0, 1))


def _tables(perm):
    a1, a2, a3 = perm
    s1_send = [r for r in range(8) if (r >> a1) & 1]
    keep1 = [r for r in range(8) if not (r >> a1) & 1]
    kidx = {r: j for j, r in enumerate(keep1)}
    s1_land = [r ^ (1 << a1) for r in s1_send]
    s2_send = [r for r in keep1 if (r >> a2) & 1]
    s2_land = [r ^ (1 << a2) for r in s2_send]
    s3_send = 1 << a3
    return s1_send, s1_land, kidx, s2_send, s2_land, s3_send


TABLES = [_tables(p) for p in PERMS]


def kernel(x, w_mat):
    def body(x_ref, w_ref, out_ref, *scr):
        keep = scr[0:3]
        staging = scr[3:6]
        recv = scr[6:9]
        ssem = scr[9:12]
        rsem = scr[12:15]
        credit = scr[15]

        me = lax.axis_index("i")
        p4 = lax.rem(me, 4)
        cz = me // 4
        cy = p4 // 2
        cx = lax.rem(p4 + cy, 2)
        partners = (
            me + 1 - 2 * lax.rem(p4, 2),
            me + 3 - 2 * p4,
            me + 4 - 8 * cz,
        )

        def block_dot(g, r):
            rx, ry, rz = r & 1, (r >> 1) & 1, (r >> 2) & 1
            qx = lax.rem(cx + rx, 2)
            qy = lax.rem(cy + ry, 2)
            qz = lax.rem(cz + rz, 2)
            q = 4 * qz + 2 * qy + lax.rem(qx + qy, 2)
            return jnp.dot(
                x_ref[pl.ds(q * M_BLK, M_BLK), :],
                w_ref[:, OFFS[g]:OFFS[g] + COLS[g]],
                preferred_element_type=jnp.float32,
            )

        def make_s1(g, i):
            return pltpu.make_async_remote_copy(
                src_ref=staging[g].at[i % 2],
                dst_ref=recv[g].at[i % 2],
                send_sem=ssem[g].at[i],
                recv_sem=rsem[g].at[i % 2],
                device_id=(partners[PERMS[g][0]],),
                device_id_type=pl.DeviceIdType.MESH,
            )

        def make_s2(g, j):
            s2_send, kidx = TABLES[g][3], TABLES[g][2]
            return pltpu.make_async_remote_copy(
                src_ref=keep[g].at[kidx[s2_send[j]]],
                dst_ref=recv[g].at[2 + j],
                send_sem=ssem[g].at[4 + j],
                recv_sem=rsem[g].at[2 + j],
                device_id=(partners[PERMS[g][1]],),
                device_id_type=pl.DeviceIdType.MESH,
            )

        def make_s3(g):
            s3_send, kidx = TABLES[g][5], TABLES[g][2]
            return pltpu.make_async_remote_copy(
                src_ref=keep[g].at[kidx[s3_send]],
                dst_ref=recv[g].at[4],
                send_sem=ssem[g].at[6],
                recv_sem=rsem[g].at[4],
                device_id=(partners[PERMS[g][2]],),
                device_id_type=pl.DeviceIdType.MESH,
            )

        barrier_sem = pltpu.get_barrier_semaphore()
        for nbr in partners:
            pl.semaphore_signal(
                barrier_sem, inc=1, device_id=(nbr,),
                device_id_type=pl.DeviceIdType.MESH,
            )
        pl.semaphore_wait(barrier_sem, 3)

        rd = {}

        for i in (0, 1):
            for g in range(3):
                staging[g][i, :, :] = block_dot(g, TABLES[g][0][i])
                rd[(g, "s1", i)] = make_s1(g, i)
                rd[(g, "s1", i)].start()

        for i in (0, 1):
            for g in range(3):
                r = TABLES[g][1][i]
                keep[g][TABLES[g][2][r], :, :] = block_dot(g, r)

        for g in range(3):
            rd[(g, "s1", 0)].wait_recv()
            k = TABLES[g][2][TABLES[g][1][0]]
            keep[g][k, :, :] = keep[g][k, :, :] + recv[g][0, :, :]
            pl.semaphore_signal(
                credit.at[g], inc=1,
                device_id=(partners[PERMS[g][0]],),
                device_id_type=pl.DeviceIdType.MESH,
            )

        for g in range(3):
            rd[(g, "s1", 0)].wait_send()
            staging[g][0, :, :] = block_dot(g, TABLES[g][0][2])
            pl.semaphore_wait(credit.at[g], 1)
            rd[(g, "s1", 2)] = make_s1(g, 2)
            rd[(g, "s1", 2)].start()

        for g in range(3):
            rd[(g, "s1", 1)].wait_recv()
            k = TABLES[g][2][TABLES[g][1][1]]
            keep[g][k, :, :] = keep[g][k, :, :] + recv[g][1, :, :]
            pl.semaphore_signal(
                credit.at[g], inc=1,
                device_id=(partners[PERMS[g][0]],),
                device_id_type=pl.DeviceIdType.MESH,
            )

        for g in range(3):
            rd[(g, "s1", 1)].wait_send()
            staging[g][1, :, :] = block_dot(g, TABLES[g][0][3])
            pl.semaphore_wait(credit.at[g], 1)
            rd[(g, "s1", 3)] = make_s1(g, 3)
            rd[(g, "s1", 3)].start()

        for i in (2, 3):
            for g in range(3):
                r = TABLES[g][1][i]
                keep[g][TABLES[g][2][r], :, :] = block_dot(g, r)

        for i in (2, 3):
            for g in range(3):
                rd[(g, "s1", i)].wait_recv()
                k = TABLES[g][2][TABLES[g][1][i]]
                keep[g][k, :, :] = keep[g][k, :, :] + recv[g][i % 2, :, :]

        for g in range(3):
            for j in (0, 1):
                rd[(g, "s2", j)] = make_s2(g, j)
                rd[(g, "s2", j)].start()
        for g in range(3):
            for j in (0, 1):
                rd[(g, "s2", j)].wait_recv()
                k = TABLES[g][2][TABLES[g][4][j]]
                keep[g][k, :, :] = keep[g][k, :, :] + recv[g][2 + j, :, :]
            rd[(g, "s3")] = make_s3(g)
            rd[(g, "s3")].start()

        for g in range(3):
            rd[(g, "s3")].wait_recv()
            out_ref[:, OFFS[g]:OFFS[g] + COLS[g]] = jnp.maximum(
                keep[g][0, :, :] + recv[g][4, :, :], 0.0
            )

        for g in range(3):
            for key in (("s1", 2), ("s1", 3), ("s2", 0), ("s2", 1), ("s3",)):
                rd[(g,) + key].wait_send()

    return pl.pallas_call(
        body,
        out_shape=jax.ShapeDtypeStruct((M_BLK, w_mat.shape[1]), jnp.float32),
        in_specs=[
            pl.BlockSpec(memory_space=pltpu.VMEM),
            pl.BlockSpec(memory_space=pltpu.VMEM),
        ],
        out_specs=pl.BlockSpec(memory_space=pltpu.VMEM),
        scratch_shapes=[
            pltpu.VMEM((4, M_BLK, COLS[0]), jnp.float32),
            pltpu.VMEM((4, M_BLK, COLS[1]), jnp.float32),
            pltpu.VMEM((4, M_BLK, COLS[2]), jnp.float32),
            pltpu.VMEM((2, M_BLK, COLS[0]), jnp.float32),
            pltpu.VMEM((2, M_BLK, COLS[1]), jnp.float32),
            pltpu.VMEM((2, M_BLK, COLS[2]), jnp.float32),
            pltpu.VMEM((5, M_BLK, COLS[0]), jnp.float32),
            pltpu.VMEM((5, M_BLK, COLS[1]), jnp.float32),
            pltpu.VMEM((5, M_BLK, COLS[2]), jnp.float32),
            pltpu.SemaphoreType.DMA((7,)),
            pltpu.SemaphoreType.DMA((7,)),
            pltpu.SemaphoreType.DMA((7,)),
            pltpu.SemaphoreType.DMA((5,)),
            pltpu.SemaphoreType.DMA((5,)),
            pltpu.SemaphoreType.DMA((5,)),
            pltpu.SemaphoreType.REGULAR((3,)),
        ],
        compiler_params=pltpu.CompilerParams(
            collective_id=0,
            vmem_limit_bytes=62 * 1024 * 1024,
        ),
    )(x, w_mat)


# device time: 139345 ns/iter; 1.2695x vs baseline; 1.0113x over previous
import jax
import jax.numpy as jnp
from jax import lax
from jax.experimental import pallas as pl
from jax.experimental.pallas import tpu as pltpu

N_DEV = 8
M_BLK = 512
COLS = (640, 640, 768)
OFFS = (0, 640, 1280)
PERMS = ((0, 1, 2), (1, 2, 0), (2, 0, 1))


def _tables(perm):
    a1, a2, a3 = perm
    s1_send = [r for r in range(8) if (r >> a1) & 1]
    keep1 = [r for r in range(8) if not (r >> a1) & 1]
    kidx = {r: j for j, r in enumerate(keep1)}
    s1_land = [r ^ (1 << a1) for r in s1_send]
    s2_send = [r for r in keep1 if (r >> a2) & 1]
    s2_land = [r ^ (1 << a2) for r in s2_send]
    s3_send = 1 << a3
    return s1_send, s1_land, kidx, s2_send, s2_land, s3_send


TABLES = [_tables(p) for p in PERMS]


def kernel(x, w_mat):
    def body(x_ref, w_ref, out_ref, *scr):
        keep = scr[0:3]
        staging = scr[3:6]
        recv = scr[6:9]
        ssem = scr[9:12]
        rsem = scr[12:15]
        credit = scr[15]

        me = lax.axis_index("i")
        p4 = lax.rem(me, 4)
        cz = me // 4
        cy = p4 // 2
        cx = lax.rem(p4 + cy, 2)
        partners = (
            me + 1 - 2 * lax.rem(p4, 2),
            me + 3 - 2 * p4,
            me + 4 - 8 * cz,
        )

        def block_dot(g, r):
            rx, ry, rz = r & 1, (r >> 1) & 1, (r >> 2) & 1
            qx = lax.rem(cx + rx, 2)
            qy = lax.rem(cy + ry, 2)
            qz = lax.rem(cz + rz, 2)
            q = 4 * qz + 2 * qy + lax.rem(qx + qy, 2)
            return jnp.dot(
                x_ref[pl.ds(q * M_BLK, M_BLK), :],
                w_ref[:, OFFS[g]:OFFS[g] + COLS[g]],
                preferred_element_type=jnp.float32,
            )

        def make_s1(g, i):
            return pltpu.make_async_remote_copy(
                src_ref=staging[g].at[i % 2],
                dst_ref=recv[g].at[i % 2],
                send_sem=ssem[g].at[i],
                recv_sem=rsem[g].at[i % 2],
                device_id=(partners[PERMS[g][0]],),
                device_id_type=pl.DeviceIdType.MESH,
            )

        def make_s2(g, j):
            s2_send, kidx = TABLES[g][3], TABLES[g][2]
            return pltpu.make_async_remote_copy(
                src_ref=keep[g].at[kidx[s2_send[j]]],
                dst_ref=recv[g].at[2 + j],
                send_sem=ssem[g].at[4 + j],
                recv_sem=rsem[g].at[2 + j],
                device_id=(partners[PERMS[g][1]],),
                device_id_type=pl.DeviceIdType.MESH,
            )

        def make_s3(g):
            s3_send, kidx = TABLES[g][5], TABLES[g][2]
            return pltpu.make_async_remote_copy(
                src_ref=keep[g].at[kidx[s3_send]],
                dst_ref=recv[g].at[4],
                send_sem=ssem[g].at[6],
                recv_sem=rsem[g].at[4],
                device_id=(partners[PERMS[g][2]],),
                device_id_type=pl.DeviceIdType.MESH,
            )

        barrier_sem = pltpu.get_barrier_semaphore()
        for nbr in partners:
            pl.semaphore_signal(
                barrier_sem, inc=1, device_id=(nbr,),
                device_id_type=pl.DeviceIdType.MESH,
            )
        pl.semaphore_wait(barrier_sem, 3)

        rd = {}
        gorder = (2, 0, 1)

        for i in (0, 1):
            for g in gorder:
                staging[g][i, :, :] = block_dot(g, TABLES[g][0][i])
                rd[(g, "s1", i)] = make_s1(g, i)
                rd[(g, "s1", i)].start()

        for i in (0, 1):
            for g in gorder:
                r = TABLES[g][1][i]
                keep[g][TABLES[g][2][r], :, :] = block_dot(g, r)

        for g in gorder:
            rd[(g, "s1", 0)].wait_recv()
            k = TABLES[g][2][TABLES[g][1][0]]
            keep[g][k, :, :] = keep[g][k, :, :] + recv[g][0, :, :]
            pl.semaphore_signal(
                credit.at[g], inc=1,
                device_id=(partners[PERMS[g][0]],),
                device_id_type=pl.DeviceIdType.MESH,
            )

        for g in gorder:
            rd[(g, "s1", 0)].wait_send()
            staging[g][0, :, :] = block_dot(g, TABLES[g][0][2])
            pl.semaphore_wait(credit.at[g], 1)
            rd[(g, "s1", 2)] = make_s1(g, 2)
            rd[(g, "s1", 2)].start()

        for g in gorder:
            rd[(g, "s1", 1)].wait_recv()
            k = TABLES[g][2][TABLES[g][1][1]]
            keep[g][k, :, :] = keep[g][k, :, :] + recv[g][1, :, :]
            pl.semaphore_signal(
                credit.at[g], inc=1,
                device_id=(partners[PERMS[g][0]],),
                device_id_type=pl.DeviceIdType.MESH,
            )

        for g in gorder:
            rd[(g, "s1", 1)].wait_send()
            staging[g][1, :, :] = block_dot(g, TABLES[g][0][3])
            pl.semaphore_wait(credit.at[g], 1)
            rd[(g, "s1", 3)] = make_s1(g, 3)
            rd[(g, "s1", 3)].start()

        for i in (2, 3):
            for g in gorder:
                r = TABLES[g][1][i]
                keep[g][TABLES[g][2][r], :, :] = block_dot(g, r)

        for g in gorder:
            for i in (2, 3):
                rd[(g, "s1", i)].wait_recv()
                k = TABLES[g][2][TABLES[g][1][i]]
                keep[g][k, :, :] = keep[g][k, :, :] + recv[g][i % 2, :, :]
            for j in (0, 1):
                rd[(g, "s2", j)] = make_s2(g, j)
                rd[(g, "s2", j)].start()

        for g in range(3):
            for j in (0, 1):
                rd[(g, "s2", j)].wait_recv()
                k = TABLES[g][2][TABLES[g][4][j]]
                keep[g][k, :, :] = keep[g][k, :, :] + recv[g][2 + j, :, :]
            rd[(g, "s3")] = make_s3(g)
            rd[(g, "s3")].start()

        for g in range(3):
            rd[(g, "s3")].wait_recv()
            out_ref[:, OFFS[g]:OFFS[g] + COLS[g]] = jnp.maximum(
                keep[g][0, :, :] + recv[g][4, :, :], 0.0
            )

        for g in range(3):
            for key in (("s1", 2), ("s1", 3), ("s2", 0), ("s2", 1), ("s3",)):
                rd[(g,) + key].wait_send()

    return pl.pallas_call(
        body,
        out_shape=jax.ShapeDtypeStruct((M_BLK, w_mat.shape[1]), jnp.float32),
        in_specs=[
            pl.BlockSpec(memory_space=pltpu.VMEM),
            pl.BlockSpec(memory_space=pltpu.VMEM),
        ],
        out_specs=pl.BlockSpec(memory_space=pltpu.VMEM),
        scratch_shapes=[
            pltpu.VMEM((4, M_BLK, COLS[0]), jnp.float32),
            pltpu.VMEM((4, M_BLK, COLS[1]), jnp.float32),
            pltpu.VMEM((4, M_BLK, COLS[2]), jnp.float32),
            pltpu.VMEM((2, M_BLK, COLS[0]), jnp.float32),
            pltpu.VMEM((2, M_BLK, COLS[1]), jnp.float32),
            pltpu.VMEM((2, M_BLK, COLS[2]), jnp.float32),
            pltpu.VMEM((5, M_BLK, COLS[0]), jnp.float32),
            pltpu.VMEM((5, M_BLK, COLS[1]), jnp.float32),
            pltpu.VMEM((5, M_BLK, COLS[2]), jnp.float32),
            pltpu.SemaphoreType.DMA((7,)),
            pltpu.SemaphoreType.DMA((7,)),
            pltpu.SemaphoreType.DMA((7,)),
            pltpu.SemaphoreType.DMA((5,)),
            pltpu.SemaphoreType.DMA((5,)),
            pltpu.SemaphoreType.DMA((5,)),
            pltpu.SemaphoreType.REGULAR((3,)),
        ],
        compiler_params=pltpu.CompilerParams(
            collective_id=0,
            vmem_limit_bytes=62 * 1024 * 1024,
        ),
    )(x, w_mat)


# device time: 127387 ns/iter; 1.3887x vs baseline; 1.0939x over previous
import jax
import jax.numpy as jnp
from jax import lax
from jax.experimental import pallas as pl
from jax.experimental.pallas import tpu as pltpu

N_DEV = 8
M_BLK = 512
COLS = (640, 640, 768)
OFFS = (0, 640, 1280)
PERMS = ((0, 1, 2), (1, 2, 0), (2, 0, 1))


def _tables(perm):
    a1, a2, a3 = perm
    keep1 = [r for r in range(8) if not (r >> a1) & 1]
    kidx = {r: j for j, r in enumerate(keep1)}
    s2_send = [(1 << a2) | (1 << a3), 1 << a2]
    s1_land = [s2_send[0], s2_send[1], 1 << a3, 0]
    s1_send = [r ^ (1 << a1) for r in s1_land]
    s2_land = [r ^ (1 << a2) for r in s2_send]
    s3_send = 1 << a3
    return s1_send, s1_land, kidx, s2_send, s2_land, s3_send


TABLES = [_tables(p) for p in PERMS]


def kernel(x, w_mat):
    def body(x_ref, w_ref, out_ref, *scr):
        keep = scr[0:3]
        staging = scr[3:6]
        recv = scr[6:9]
        ssem = scr[9:12]
        rsem = scr[12:15]
        credit = scr[15]

        me = lax.axis_index("i")
        p4 = lax.rem(me, 4)
        cz = me // 4
        cy = p4 // 2
        cx = lax.rem(p4 + cy, 2)
        partners = (
            me + 1 - 2 * lax.rem(p4, 2),
            me + 3 - 2 * p4,
            me + 4 - 8 * cz,
        )

        def block_dot(g, r):
            rx, ry, rz = r & 1, (r >> 1) & 1, (r >> 2) & 1
            qx = lax.rem(cx + rx, 2)
            qy = lax.rem(cy + ry, 2)
            qz = lax.rem(cz + rz, 2)
            q = 4 * qz + 2 * qy + lax.rem(qx + qy, 2)
            return jnp.dot(
                x_ref[pl.ds(q * M_BLK, M_BLK), :],
                w_ref[:, OFFS[g]:OFFS[g] + COLS[g]],
                preferred_element_type=jnp.float32,
            )

        def make_s1(g, i):
            return pltpu.make_async_remote_copy(
                src_ref=staging[g].at[i % 2],
                dst_ref=recv[g].at[i % 2],
                send_sem=ssem[g].at[i],
                recv_sem=rsem[g].at[i % 2],
                device_id=(partners[PERMS[g][0]],),
                device_id_type=pl.DeviceIdType.MESH,
            )

        def make_s2(g, j):
            s2_send, kidx = TABLES[g][3], TABLES[g][2]
            return pltpu.make_async_remote_copy(
                src_ref=keep[g].at[kidx[s2_send[j]]],
                dst_ref=recv[g].at[2 + j],
                send_sem=ssem[g].at[4 + j],
                recv_sem=rsem[g].at[2 + j],
                device_id=(partners[PERMS[g][1]],),
                device_id_type=pl.DeviceIdType.MESH,
            )

        def make_s3(g):
            s3_send, kidx = TABLES[g][5], TABLES[g][2]
            return pltpu.make_async_remote_copy(
                src_ref=keep[g].at[kidx[s3_send]],
                dst_ref=recv[g].at[4],
                send_sem=ssem[g].at[6],
                recv_sem=rsem[g].at[4],
                device_id=(partners[PERMS[g][2]],),
                device_id_type=pl.DeviceIdType.MESH,
            )

        barrier_sem = pltpu.get_barrier_semaphore()
        for nbr in partners:
            pl.semaphore_signal(
                barrier_sem, inc=1, device_id=(nbr,),
                device_id_type=pl.DeviceIdType.MESH,
            )
        pl.semaphore_wait(barrier_sem, 3)

        rd = {}
        gorder = (2, 0, 1)

        for i in (0, 1):
            for g in gorder:
                staging[g][i, :, :] = block_dot(g, TABLES[g][0][i])
                rd[(g, "s1", i)] = make_s1(g, i)
                rd[(g, "s1", i)].start()

        for i in (0, 1):
            for g in gorder:
                r = TABLES[g][1][i]
                keep[g][TABLES[g][2][r], :, :] = block_dot(g, r)

        for g in gorder:
            rd[(g, "s1", 0)].wait_recv()
            k = TABLES[g][2][TABLES[g][1][0]]
            keep[g][k, :, :] = keep[g][k, :, :] + recv[g][0, :, :]
            pl.semaphore_signal(
                credit.at[g], inc=1,
                device_id=(partners[PERMS[g][0]],),
                device_id_type=pl.DeviceIdType.MESH,
            )

        for g in gorder:
            rd[(g, "s1", 0)].wait_send()
            staging[g][0, :, :] = block_dot(g, TABLES[g][0][2])
            pl.semaphore_wait(credit.at[g], 1)
            rd[(g, "s1", 2)] = make_s1(g, 2)
            rd[(g, "s1", 2)].start()

        for g in gorder:
            rd[(g, "s2", 0)] = make_s2(g, 0)
            rd[(g, "s2", 0)].start()

        for g in gorder:
            r = TABLES[g][1][2]
            keep[g][TABLES[g][2][r], :, :] = block_dot(g, r)

        for g in gorder:
            rd[(g, "s1", 1)].wait_recv()
            k = TABLES[g][2][TABLES[g][1][1]]
            keep[g][k, :, :] = keep[g][k, :, :] + recv[g][1, :, :]
            pl.semaphore_signal(
                credit.at[g], inc=1,
                device_id=(partners[PERMS[g][0]],),
                device_id_type=pl.DeviceIdType.MESH,
            )

        for g in gorder:
            rd[(g, "s1", 1)].wait_send()
            staging[g][1, :, :] = block_dot(g, TABLES[g][0][3])
            pl.semaphore_wait(credit.at[g], 1)
            rd[(g, "s1", 3)] = make_s1(g, 3)
            rd[(g, "s1", 3)].start()

        for g in gorder:
            rd[(g, "s2", 1)] = make_s2(g, 1)
            rd[(g, "s2", 1)].start()

        for g in gorder:
            r = TABLES[g][1][3]
            keep[g][TABLES[g][2][r], :, :] = block_dot(g, r)

        for g in gorder:
            k3 = TABLES[g][2][TABLES[g][5]]
            rd[(g, "s1", 2)].wait_recv()
            keep[g][k3, :, :] = keep[g][k3, :, :] + recv[g][0, :, :]
            rd[(g, "s2", 0)].wait_recv()
            keep[g][k3, :, :] = keep[g][k3, :, :] + recv[g][2, :, :]
            rd[(g, "s3")] = make_s3(g)
            rd[(g, "s3")].start()

        for g in gorder:
            rd[(g, "s1", 3)].wait_recv()
            keep[g][0, :, :] = keep[g][0, :, :] + recv[g][1, :, :]
            rd[(g, "s2", 1)].wait_recv()
            keep[g][0, :, :] = keep[g][0, :, :] + recv[g][3, :, :]

        for g in gorder:
            rd[(g, "s3")].wait_recv()
            out_ref[:, OFFS[g]:OFFS[g] + COLS[g]] = jnp.maximum(
                keep[g][0, :, :] + recv[g][4, :, :], 0.0
            )

        for g in range(3):
            for key in (("s1", 2), ("s1", 3), ("s2", 0), ("s2", 1), ("s3",)):
                rd[(g,) + key].wait_send()

    return pl.pallas_call(
        body,
        out_shape=jax.ShapeDtypeStruct((M_BLK, w_mat.shape[1]), jnp.float32),
        in_specs=[
            pl.BlockSpec(memory_space=pltpu.VMEM),
            pl.BlockSpec(memory_space=pltpu.VMEM),
        ],
        out_specs=pl.BlockSpec(memory_space=pltpu.VMEM),
        scratch_shapes=[
            pltpu.VMEM((4, M_BLK, COLS[0]), jnp.float32),
            pltpu.VMEM((4, M_BLK, COLS[1]), jnp.float32),
            pltpu.VMEM((4, M_BLK, COLS[2]), jnp.float32),
            pltpu.VMEM((2, M_BLK, COLS[0]), jnp.float32),
            pltpu.VMEM((2, M_BLK, COLS[1]), jnp.float32),
            pltpu.VMEM((2, M_BLK, COLS[2]), jnp.float32),
            pltpu.VMEM((5, M_BLK, COLS[0]), jnp.float32),
            pltpu.VMEM((5, M_BLK, COLS[1]), jnp.float32),
            pltpu.VMEM((5, M_BLK, COLS[2]), jnp.float32),
            pltpu.SemaphoreType.DMA((7,)),
            pltpu.SemaphoreType.DMA((7,)),
            pltpu.SemaphoreType.DMA((7,)),
            pltpu.SemaphoreType.DMA((5,)),
            pltpu.SemaphoreType.DMA((5,)),
            pltpu.SemaphoreType.DMA((5,)),
            pltpu.SemaphoreType.REGULAR((3,)),
        ],
        compiler_params=pltpu.CompilerParams(
            collective_id=0,
            vmem_limit_bytes=62 * 1024 * 1024,
        ),
    )(x, w_mat)
